# Initial kernel scaffold; baseline (speedup 1.0000x reference)
import jax
import jax.numpy as jnp
from jax.experimental import pallas as pl


def kernel(x, edge_index, W1, b1, W2, b2, Wfc, bfc):
    raise NotImplementedError("write your pallas kernel here")


# speedup vs baseline: 7.7065x; 7.7065x over previous
import functools

import jax
import jax.numpy as jnp
from jax import lax
from jax.experimental import pallas as pl
from jax.experimental.pallas import tpu as pltpu
from jax.experimental.pallas import tpu_sc as plsc

N_NODES_C = 10000
N_EDGES_C = 160000
L = 16
NV = N_NODES_C // L
CH = 2000
NCHUNK = N_EDGES_C // CH
UNROLL = 5
NINNER = (CH // L) // UNROLL


def _rsqrt16(x):
    i = plsc.bitcast(x, jnp.int32)
    i = jnp.int32(0x5F3759DF) - lax.shift_right_logical(i, 1)
    y = plsc.bitcast(i, jnp.float32)
    for _ in range(3):
        y = y * (1.5 - 0.5 * x * y * y)
    return y


def _sc_mesh():
    return plsc.VectorSubcoreMesh(
        core_axis_name="c", subcore_axis_name="s", num_cores=2, num_subcores=16
    )


def _agg_phase(src_hbm, dst_hbm, dinv_v, hs_v, acc_v, src_v, dst_v, b_v,
               zT_hbm, f):
    def scale(i, _):
        s = pl.ds(i * L, L)
        h = hs_v[s] * dinv_v[s]
        hs_v[s] = h
        acc_v[s] = h
        return 0

    lax.fori_loop(0, NV, scale, 0)

    def edgechunk(k, _):
        pltpu.sync_copy(src_hbm.at[pl.ds(k * CH, CH)], src_v)
        pltpu.sync_copy(dst_hbm.at[pl.ds(k * CH, CH)], dst_v)

        def inner(i, _):
            for u in range(UNROLL):
                o = (i * UNROLL + u) * L
                s16 = src_v[pl.ds(o, L)]
                d16 = dst_v[pl.ds(o, L)]
                hv = plsc.load_gather(hs_v, [s16])
                plsc.addupdate_scatter(acc_v, [d16], hv)
            return 0

        lax.fori_loop(0, NINNER, inner, 0)
        return 0

    lax.fori_loop(0, NCHUNK, edgechunk, 0)

    bvec = plsc.load_gather(b_v, [jnp.full((L,), f, jnp.int32)])

    def fin(i, _):
        s = pl.ds(i * L, L)
        acc_v[s] = jnp.maximum(acc_v[s] * dinv_v[s] + bvec, 0.0)
        return 0

    lax.fori_loop(0, NV, fin, 0)
    pltpu.sync_copy(acc_v, zT_hbm.at[f])


def _sc_layer1_body(hT_hbm, src_hbm, dst_hbm, b_hbm, zT_hbm, dinv_hbm,
                    dinv_v, hs_v, acc_v, src_v, dst_v, b_v):
    cid = lax.axis_index("c")
    sid = lax.axis_index("s")
    wid = sid * 2 + cid

    def initb(i, _):
        dinv_v[pl.ds(i * L, L)] = jnp.full((L,), 1.0, jnp.float32)
        return 0

    lax.fori_loop(0, NV, initb, 0)
    ones = jnp.full((L,), 1.0, jnp.float32)

    def degchunk(k, _):
        pltpu.sync_copy(dst_hbm.at[pl.ds(k * CH, CH)], dst_v)

        def inner(i, _):
            for u in range(UNROLL):
                d16 = dst_v[pl.ds((i * UNROLL + u) * L, L)]
                plsc.addupdate_scatter(dinv_v, [d16], ones)
            return 0

        lax.fori_loop(0, NINNER, inner, 0)
        return 0

    lax.fori_loop(0, NCHUNK, degchunk, 0)

    def rs(i, _):
        s = pl.ds(i * L, L)
        dinv_v[s] = _rsqrt16(dinv_v[s])
        return 0

    lax.fori_loop(0, NV, rs, 0)

    @pl.when(wid == 31)
    def _():
        pltpu.sync_copy(dinv_v, dinv_hbm)

    @pl.when(wid < 24)
    def _():
        pltpu.sync_copy(hT_hbm.at[wid], hs_v)
        pltpu.sync_copy(b_hbm, b_v)
        _agg_phase(src_hbm, dst_hbm, dinv_v, hs_v, acc_v, src_v, dst_v, b_v,
                   zT_hbm, wid)


def _sc_layer2_body(hT_hbm, src_hbm, dst_hbm, b_hbm, dinv_hbm, zT_hbm,
                    dinv_v, hs_v, acc_v, src_v, dst_v, b_v):
    cid = lax.axis_index("c")
    sid = lax.axis_index("s")
    wid = sid * 2 + cid

    @pl.when(wid < 16)
    def _():
        pltpu.sync_copy(dinv_hbm, dinv_v)
        pltpu.sync_copy(hT_hbm.at[wid], hs_v)
        pltpu.sync_copy(b_hbm, b_v)
        _agg_phase(src_hbm, dst_hbm, dinv_v, hs_v, acc_v, src_v, dst_v, b_v,
                   zT_hbm, wid)


def _sc_scratch():
    return [
        pltpu.VMEM((N_NODES_C,), jnp.float32),
        pltpu.VMEM((N_NODES_C,), jnp.float32),
        pltpu.VMEM((N_NODES_C,), jnp.float32),
        pltpu.VMEM((CH,), jnp.int32),
        pltpu.VMEM((CH,), jnp.int32),
        pltpu.VMEM((32,), jnp.float32),
    ]


def _sc_layer1(hT, src, dst, bpad):
    f = pl.kernel(
        _sc_layer1_body,
        out_type=(
            jax.ShapeDtypeStruct((24, N_NODES_C), jnp.float32),
            jax.ShapeDtypeStruct((N_NODES_C,), jnp.float32),
        ),
        mesh=_sc_mesh(),
        scratch_types=_sc_scratch(),
        compiler_params=pltpu.CompilerParams(needs_layout_passes=False),
    )
    return f(hT, src, dst, bpad)


def _sc_layer2(hT, src, dst, bpad, dinv):
    f = pl.kernel(
        _sc_layer2_body,
        out_type=jax.ShapeDtypeStruct((16, N_NODES_C), jnp.float32),
        mesh=_sc_mesh(),
        scratch_types=_sc_scratch(),
        compiler_params=pltpu.CompilerParams(needs_layout_passes=False),
    )
    return f(hT, src, dst, bpad, dinv)


def _mm_body(x_ref, w_ref, o_ref):
    o_ref[...] = jnp.dot(x_ref[...], w_ref[...],
                         preferred_element_type=jnp.float32)


def _tc_matmul(x, w):
    return pl.pallas_call(
        _mm_body,
        out_shape=jax.ShapeDtypeStruct((x.shape[0], w.shape[1]), jnp.float32),
    )(x, w)


def _fc_body(x_ref, w_ref, b_ref, o_ref):
    o_ref[...] = jnp.maximum(
        jnp.dot(x_ref[...], w_ref[...], preferred_element_type=jnp.float32)
        + b_ref[...],
        0.0,
    )


def _tc_fc(x, w, b2d):
    return pl.pallas_call(
        _fc_body,
        out_shape=jax.ShapeDtypeStruct((x.shape[0], w.shape[1]), jnp.float32),
    )(x, w, b2d)


def kernel(x, edge_index, W1, b1, W2, b2, Wfc, bfc):
    ei = edge_index.astype(jnp.int32)
    src = ei[0]
    dst = ei[1]
    b1p = jnp.zeros((32,), jnp.float32).at[: b1.shape[0]].set(b1)
    b2p = jnp.zeros((32,), jnp.float32).at[: b2.shape[0]].set(b2)

    h1 = _tc_matmul(x, W1)
    z1T, dinv = _sc_layer1(h1.T, src, dst, b1p)
    h2T = _tc_matmul(W2.T, z1T)
    z2T = _sc_layer2(h2T, src, dst, b2p, dinv)
    out = _tc_fc(z2T.T, Wfc, bfc.reshape(1, -1))
    return out


# speedup vs baseline: 13.3828x; 1.7366x over previous
import functools

import jax
import jax.numpy as jnp
from jax import lax
from jax.experimental import pallas as pl
from jax.experimental.pallas import tpu as pltpu
from jax.experimental.pallas import tpu_sc as plsc

N_NODES_C = 10000
N_EDGES_C = 160000
L = 16
NV = N_NODES_C // L
CH = 8000
NCHUNK = N_EDGES_C // CH
UNROLL = 5
NINNER = (CH // L) // UNROLL
NBUF = 2


def _rsqrt16(x):
    i = plsc.bitcast(x, jnp.int32)
    i = jnp.int32(0x5F3759DF) - lax.shift_right_logical(i, 1)
    y = plsc.bitcast(i, jnp.float32)
    for _ in range(3):
        y = y * (1.5 - 0.5 * x * y * y)
    return y


def _sc_mesh():
    return plsc.VectorSubcoreMesh(
        core_axis_name="c", subcore_axis_name="s", num_cores=2, num_subcores=16
    )


def _start_chunk(hbm, buf_ref, sem_ref, k):
    pltpu.make_async_copy(hbm.at[pl.ds(k * CH, CH)], buf_ref, sem_ref).start()


def _wait_chunk(hbm, buf_ref, sem_ref, k):
    pltpu.make_async_copy(hbm.at[pl.ds(k * CH, CH)], buf_ref, sem_ref).wait()


def _edge_ring(streams, compute):
    for hbm, bufs, sems in streams:
        _start_chunk(hbm, bufs[0], sems[0], 0)

    def outer(i, _):
        for b in range(NBUF):
            k = i * NBUF + b
            nk = k + 1

            @pl.when(nk < NCHUNK)
            def _():
                for hbm, bufs, sems in streams:
                    _start_chunk(hbm, bufs[1 - b], sems[1 - b], nk)

            for hbm, bufs, sems in streams:
                _wait_chunk(hbm, bufs[b], sems[b], k)
            compute([bufs[b] for _, bufs, _s in streams], k)
        return 0

    lax.fori_loop(0, NCHUNK // NBUF, outer, 0)


def _agg_phase(src_hbm, dst_hbm, dinv_v, hs_v, acc_v, src_bufs, dst_bufs,
               b_v, sem_s, sem_d, zT_hbm, f):
    def scale(i, _):
        s = pl.ds(i * L, L)
        h = hs_v[s] * dinv_v[s]
        hs_v[s] = h
        acc_v[s] = h
        return 0

    lax.fori_loop(0, NV, scale, 0)

    def compute(bufs, k):
        sref, dref = bufs

        def inner(i, _):
            for u in range(UNROLL):
                o = (i * UNROLL + u) * L
                s16 = sref[pl.ds(o, L)]
                d16 = dref[pl.ds(o, L)]
                hv = plsc.load_gather(hs_v, [s16])
                plsc.addupdate_scatter(acc_v, [d16], hv)
            return 0

        lax.fori_loop(0, NINNER, inner, 0)

    _edge_ring(((src_hbm, src_bufs, sem_s), (dst_hbm, dst_bufs, sem_d)),
               compute)

    bvec = plsc.load_gather(b_v, [jnp.full((L,), f, jnp.int32)])

    def fin(i, _):
        s = pl.ds(i * L, L)
        acc_v[s] = jnp.maximum(acc_v[s] * dinv_v[s] + bvec, 0.0)
        return 0

    lax.fori_loop(0, NV, fin, 0)
    pltpu.sync_copy(acc_v, zT_hbm.at[f])


def _sc_layer1_body(hT_hbm, src_hbm, dst_hbm, b_hbm, zT_hbm, dinv_hbm,
                    dinv_v, hs_v, acc_v, src_v0, src_v1, dst_v0, dst_v1, b_v,
                    sem_s0, sem_s1, sem_d0, sem_d1, sem_h):
    src_bufs = (src_v0, src_v1)
    dst_bufs = (dst_v0, dst_v1)
    sem_s = (sem_s0, sem_s1)
    sem_d = (sem_d0, sem_d1)
    cid = lax.axis_index("c")
    sid = lax.axis_index("s")
    wid = sid * 2 + cid

    @pl.when(wid < 24)
    def _():
        pltpu.make_async_copy(hT_hbm.at[wid], hs_v, sem_h).start()

    def initb(i, _):
        dinv_v[pl.ds(i * L, L)] = jnp.full((L,), 1.0, jnp.float32)
        return 0

    lax.fori_loop(0, NV, initb, 0)
    ones = jnp.full((L,), 1.0, jnp.float32)

    def degcompute(bufs, k):
        dref = bufs[0]

        def inner(i, _):
            for u in range(UNROLL):
                d16 = dref[pl.ds((i * UNROLL + u) * L, L)]
                plsc.addupdate_scatter(dinv_v, [d16], ones)
            return 0

        lax.fori_loop(0, NINNER, inner, 0)

    _edge_ring(((dst_hbm, dst_bufs, sem_d),), degcompute)

    def rs(i, _):
        s = pl.ds(i * L, L)
        dinv_v[s] = _rsqrt16(dinv_v[s])
        return 0

    lax.fori_loop(0, NV, rs, 0)

    @pl.when(wid == 31)
    def _():
        pltpu.sync_copy(dinv_v, dinv_hbm)

    @pl.when(wid < 24)
    def _():
        pltpu.make_async_copy(hT_hbm.at[wid], hs_v, sem_h).wait()
        pltpu.sync_copy(b_hbm, b_v)
        _agg_phase(src_hbm, dst_hbm, dinv_v, hs_v, acc_v, src_bufs, dst_bufs,
                   b_v, sem_s, sem_d, zT_hbm, wid)


def _sc_layer2_body(hT_hbm, src_hbm, dst_hbm, b_hbm, dinv_hbm, zT_hbm,
                    dinv_v, hs_v, acc_v, src_v0, src_v1, dst_v0, dst_v1, b_v,
                    sem_s0, sem_s1, sem_d0, sem_d1, sem_h):
    src_bufs = (src_v0, src_v1)
    dst_bufs = (dst_v0, dst_v1)
    sem_s = (sem_s0, sem_s1)
    sem_d = (sem_d0, sem_d1)
    cid = lax.axis_index("c")
    sid = lax.axis_index("s")
    wid = sid * 2 + cid

    @pl.when(wid < 16)
    def _():
        pltpu.make_async_copy(hT_hbm.at[wid], hs_v, sem_h).start()
        pltpu.sync_copy(dinv_hbm, dinv_v)
        pltpu.sync_copy(b_hbm, b_v)
        pltpu.make_async_copy(hT_hbm.at[wid], hs_v, sem_h).wait()
        _agg_phase(src_hbm, dst_hbm, dinv_v, hs_v, acc_v, src_bufs, dst_bufs,
                   b_v, sem_s, sem_d, zT_hbm, wid)


def _sc_scratch():
    return [
        pltpu.VMEM((N_NODES_C,), jnp.float32),
        pltpu.VMEM((N_NODES_C,), jnp.float32),
        pltpu.VMEM((N_NODES_C,), jnp.float32),
        pltpu.VMEM((CH,), jnp.int32),
        pltpu.VMEM((CH,), jnp.int32),
        pltpu.VMEM((CH,), jnp.int32),
        pltpu.VMEM((CH,), jnp.int32),
        pltpu.VMEM((32,), jnp.float32),
        pltpu.SemaphoreType.DMA,
        pltpu.SemaphoreType.DMA,
        pltpu.SemaphoreType.DMA,
        pltpu.SemaphoreType.DMA,
        pltpu.SemaphoreType.DMA,
    ]


def _sc_layer1(hT, src, dst, bpad):
    f = pl.kernel(
        _sc_layer1_body,
        out_type=(
            jax.ShapeDtypeStruct((24, N_NODES_C), jnp.float32),
            jax.ShapeDtypeStruct((N_NODES_C,), jnp.float32),
        ),
        mesh=_sc_mesh(),
        scratch_types=_sc_scratch(),
        compiler_params=pltpu.CompilerParams(needs_layout_passes=False),
    )
    return f(hT, src, dst, bpad)


def _sc_layer2(hT, src, dst, bpad, dinv):
    f = pl.kernel(
        _sc_layer2_body,
        out_type=jax.ShapeDtypeStruct((16, N_NODES_C), jnp.float32),
        mesh=_sc_mesh(),
        scratch_types=_sc_scratch(),
        compiler_params=pltpu.CompilerParams(needs_layout_passes=False),
    )
    return f(hT, src, dst, bpad, dinv)


def _mm_body(x_ref, w_ref, o_ref):
    o_ref[...] = jnp.dot(x_ref[...], w_ref[...],
                         preferred_element_type=jnp.float32)


def _tc_matmul(x, w):
    return pl.pallas_call(
        _mm_body,
        out_shape=jax.ShapeDtypeStruct((x.shape[0], w.shape[1]), jnp.float32),
    )(x, w)


def _fc_body(x_ref, w_ref, b_ref, o_ref):
    o_ref[...] = jnp.maximum(
        jnp.dot(x_ref[...], w_ref[...], preferred_element_type=jnp.float32)
        + b_ref[...],
        0.0,
    )


def _tc_fc(x, w, b2d):
    return pl.pallas_call(
        _fc_body,
        out_shape=jax.ShapeDtypeStruct((x.shape[0], w.shape[1]), jnp.float32),
    )(x, w, b2d)


def kernel(x, edge_index, W1, b1, W2, b2, Wfc, bfc):
    ei = edge_index.astype(jnp.int32)
    src = ei[0]
    dst = ei[1]
    b1p = jnp.zeros((32,), jnp.float32).at[: b1.shape[0]].set(b1)
    b2p = jnp.zeros((32,), jnp.float32).at[: b2.shape[0]].set(b2)

    h1 = _tc_matmul(x, W1)
    z1T, dinv = _sc_layer1(h1.T, src, dst, b1p)
    h2T = _tc_matmul(W2.T, z1T)
    z2T = _sc_layer2(h2T, src, dst, b2p, dinv)
    out = _tc_fc(z2T.T, Wfc, bfc.reshape(1, -1))
    return out


# speedup vs baseline: 29.1581x; 2.1788x over previous
import functools

import jax
import jax.numpy as jnp
from jax import lax
from jax.experimental import pallas as pl
from jax.experimental.pallas import tpu as pltpu
from jax.experimental.pallas import tpu_sc as plsc

N_NODES_C = 10000
N_EDGES_C = 160000
L = 16
NV = N_NODES_C // L
CH = 8000
NCHUNK = N_EDGES_C // CH
UNROLL = 5
NINNER = (CH // L) // UNROLL
NBUF = 2


def _rsqrt16(x):
    i = plsc.bitcast(x, jnp.int32)
    i = jnp.int32(0x5F3759DF) - lax.shift_right_logical(i, 1)
    y = plsc.bitcast(i, jnp.float32)
    for _ in range(3):
        y = y * (1.5 - 0.5 * x * y * y)
    return y


def _sc_mesh():
    return plsc.VectorSubcoreMesh(
        core_axis_name="c", subcore_axis_name="s", num_cores=2, num_subcores=16
    )


def _start_chunk(hbm, buf_ref, sem_ref, k):
    pltpu.make_async_copy(hbm.at[pl.ds(k * CH, CH)], buf_ref, sem_ref).start()


def _wait_chunk(hbm, buf_ref, sem_ref, k):
    pltpu.make_async_copy(hbm.at[pl.ds(k * CH, CH)], buf_ref, sem_ref).wait()


def _edge_ring(streams, compute):
    for hbm, bufs, sems in streams:
        _start_chunk(hbm, bufs[0], sems[0], 0)

    def outer(i, _):
        for b in range(NBUF):
            k = i * NBUF + b
            nk = k + 1

            @pl.when(nk < NCHUNK)
            def _():
                for hbm, bufs, sems in streams:
                    _start_chunk(hbm, bufs[1 - b], sems[1 - b], nk)

            for hbm, bufs, sems in streams:
                _wait_chunk(hbm, bufs[b], sems[b], k)
            compute([bufs[b] for _, bufs, _s in streams], k)
        return 0

    lax.fori_loop(0, NCHUNK // NBUF, outer, 0)


def _agg_phase(src_hbm, dst_hbm, dinv_v, hs_v, acc_v, src_bufs, dst_bufs,
               b_v, sem_s, sem_d, zT_hbm, f):
    @plsc.parallel_loop(0, N_NODES_C, L, unroll=4)
    def _(o):
        s = pl.ds(o, L)
        h = hs_v[s] * dinv_v[s]
        hs_v[s] = h
        acc_v[s] = h

    def compute(bufs, k):
        sref, dref = bufs

        @plsc.parallel_loop(0, CH, L, unroll=UNROLL)
        def _(o):
            s16 = sref[pl.ds(o, L)]
            d16 = dref[pl.ds(o, L)]
            hv = plsc.load_gather(hs_v, [s16])
            plsc.addupdate_scatter(acc_v, [d16], hv)

    _edge_ring(((src_hbm, src_bufs, sem_s), (dst_hbm, dst_bufs, sem_d)),
               compute)

    bvec = plsc.load_gather(b_v, [jnp.full((L,), f, jnp.int32)])

    @plsc.parallel_loop(0, N_NODES_C, L, unroll=4)
    def _(o):
        s = pl.ds(o, L)
        acc_v[s] = jnp.maximum(acc_v[s] * dinv_v[s] + bvec, 0.0)

    pltpu.sync_copy(acc_v, zT_hbm.at[f])


def _sc_layer1_body(hT_hbm, src_hbm, dst_hbm, b_hbm, zT_hbm, dinv_hbm,
                    dinv_v, hs_v, acc_v, src_v0, src_v1, dst_v0, dst_v1, b_v,
                    sem_s0, sem_s1, sem_d0, sem_d1, sem_h):
    src_bufs = (src_v0, src_v1)
    dst_bufs = (dst_v0, dst_v1)
    sem_s = (sem_s0, sem_s1)
    sem_d = (sem_d0, sem_d1)
    cid = lax.axis_index("c")
    sid = lax.axis_index("s")
    wid = sid * 2 + cid

    @pl.when(wid < 24)
    def _():
        pltpu.make_async_copy(hT_hbm.at[wid], hs_v, sem_h).start()

    @plsc.parallel_loop(0, N_NODES_C, L, unroll=4)
    def _(o):
        dinv_v[pl.ds(o, L)] = jnp.full((L,), 1.0, jnp.float32)

    ones = jnp.full((L,), 1.0, jnp.float32)

    def degcompute(bufs, k):
        dref = bufs[0]

        @plsc.parallel_loop(0, CH, L, unroll=UNROLL)
        def _(o):
            d16 = dref[pl.ds(o, L)]
            plsc.addupdate_scatter(dinv_v, [d16], ones)

    _edge_ring(((dst_hbm, dst_bufs, sem_d),), degcompute)

    @plsc.parallel_loop(0, N_NODES_C, L, unroll=4)
    def _(o):
        s = pl.ds(o, L)
        dinv_v[s] = _rsqrt16(dinv_v[s])

    @pl.when(wid == 31)
    def _():
        pltpu.sync_copy(dinv_v, dinv_hbm)

    @pl.when(wid < 24)
    def _():
        pltpu.make_async_copy(hT_hbm.at[wid], hs_v, sem_h).wait()
        pltpu.sync_copy(b_hbm, b_v)
        _agg_phase(src_hbm, dst_hbm, dinv_v, hs_v, acc_v, src_bufs, dst_bufs,
                   b_v, sem_s, sem_d, zT_hbm, wid)


def _sc_layer2_body(hT_hbm, src_hbm, dst_hbm, b_hbm, dinv_hbm, zT_hbm,
                    dinv_v, hs_v, acc_v, src_v0, src_v1, dst_v0, dst_v1, b_v,
                    sem_s0, sem_s1, sem_d0, sem_d1, sem_h):
    src_bufs = (src_v0, src_v1)
    dst_bufs = (dst_v0, dst_v1)
    sem_s = (sem_s0, sem_s1)
    sem_d = (sem_d0, sem_d1)
    cid = lax.axis_index("c")
    sid = lax.axis_index("s")
    wid = sid * 2 + cid

    @pl.when(wid < 16)
    def _():
        pltpu.make_async_copy(hT_hbm.at[wid], hs_v, sem_h).start()
        pltpu.sync_copy(dinv_hbm, dinv_v)
        pltpu.sync_copy(b_hbm, b_v)
        pltpu.make_async_copy(hT_hbm.at[wid], hs_v, sem_h).wait()
        _agg_phase(src_hbm, dst_hbm, dinv_v, hs_v, acc_v, src_bufs, dst_bufs,
                   b_v, sem_s, sem_d, zT_hbm, wid)


def _sc_scratch():
    return [
        pltpu.VMEM((N_NODES_C,), jnp.float32),
        pltpu.VMEM((N_NODES_C,), jnp.float32),
        pltpu.VMEM((N_NODES_C,), jnp.float32),
        pltpu.VMEM((CH,), jnp.int32),
        pltpu.VMEM((CH,), jnp.int32),
        pltpu.VMEM((CH,), jnp.int32),
        pltpu.VMEM((CH,), jnp.int32),
        pltpu.VMEM((32,), jnp.float32),
        pltpu.SemaphoreType.DMA,
        pltpu.SemaphoreType.DMA,
        pltpu.SemaphoreType.DMA,
        pltpu.SemaphoreType.DMA,
        pltpu.SemaphoreType.DMA,
    ]


def _sc_layer1(hT, src, dst, bpad):
    f = pl.kernel(
        _sc_layer1_body,
        out_type=(
            jax.ShapeDtypeStruct((24, N_NODES_C), jnp.float32),
            jax.ShapeDtypeStruct((N_NODES_C,), jnp.float32),
        ),
        mesh=_sc_mesh(),
        scratch_types=_sc_scratch(),
        compiler_params=pltpu.CompilerParams(needs_layout_passes=False),
    )
    return f(hT, src, dst, bpad)


def _sc_layer2(hT, src, dst, bpad, dinv):
    f = pl.kernel(
        _sc_layer2_body,
        out_type=jax.ShapeDtypeStruct((16, N_NODES_C), jnp.float32),
        mesh=_sc_mesh(),
        scratch_types=_sc_scratch(),
        compiler_params=pltpu.CompilerParams(needs_layout_passes=False),
    )
    return f(hT, src, dst, bpad, dinv)


def _mm_body(x_ref, w_ref, o_ref):
    o_ref[...] = jnp.dot(x_ref[...], w_ref[...],
                         preferred_element_type=jnp.float32)


def _tc_matmul(x, w):
    return pl.pallas_call(
        _mm_body,
        out_shape=jax.ShapeDtypeStruct((x.shape[0], w.shape[1]), jnp.float32),
    )(x, w)


def _fc_body(x_ref, w_ref, b_ref, o_ref):
    o_ref[...] = jnp.maximum(
        jnp.dot(x_ref[...], w_ref[...], preferred_element_type=jnp.float32)
        + b_ref[...],
        0.0,
    )


def _tc_fc(x, w, b2d):
    return pl.pallas_call(
        _fc_body,
        out_shape=jax.ShapeDtypeStruct((x.shape[0], w.shape[1]), jnp.float32),
    )(x, w, b2d)


def kernel(x, edge_index, W1, b1, W2, b2, Wfc, bfc):
    ei = edge_index.astype(jnp.int32)
    src = ei[0]
    dst = ei[1]
    b1p = jnp.zeros((32,), jnp.float32).at[: b1.shape[0]].set(b1)
    b2p = jnp.zeros((32,), jnp.float32).at[: b2.shape[0]].set(b2)

    h1 = _tc_matmul(x, W1)
    z1T, dinv = _sc_layer1(h1.T, src, dst, b1p)
    h2T = _tc_matmul(W2.T, z1T)
    z2T = _sc_layer2(h2T, src, dst, b2p, dinv)
    out = _tc_fc(z2T.T, Wfc, bfc.reshape(1, -1))
    return out


# speedup vs baseline: 30.3459x; 1.0407x over previous
import functools

import jax
import jax.numpy as jnp
from jax import lax
from jax.experimental import pallas as pl
from jax.experimental.pallas import tpu as pltpu
from jax.experimental.pallas import tpu_sc as plsc

N_NODES_C = 10000
N_EDGES_C = 160000
L = 16
CH = 4000
NCHUNK_ALL = N_EDGES_C // CH
NTCH = 30
UNROLL = 5
NBUF = 2
FSC = 12


def _rsqrt16(x):
    i = plsc.bitcast(x, jnp.int32)
    i = jnp.int32(0x5F3759DF) - lax.shift_right_logical(i, 1)
    y = plsc.bitcast(i, jnp.float32)
    for _ in range(3):
        y = y * (1.5 - 0.5 * x * y * y)
    return y


def _sc_mesh():
    return plsc.VectorSubcoreMesh(
        core_axis_name="c", subcore_axis_name="s", num_cores=2, num_subcores=16
    )


def _ring(n_chunks, base_of, streams, compute):
    def start(j, b):
        for hbm, bufs, sems in streams:
            pltpu.make_async_copy(
                hbm.at[pl.ds(base_of(j), CH)], bufs[b], sems[b]
            ).start()

    def wait(j, b):
        for hbm, bufs, sems in streams:
            pltpu.make_async_copy(
                hbm.at[pl.ds(base_of(j), CH)], bufs[b], sems[b]
            ).wait()

    start(0, 0)

    def outer(i, _):
        for b in range(NBUF):
            j = i * NBUF + b
            nj = j + 1

            @pl.when(nj < n_chunks)
            def _():
                start(nj, 1 - b)

            wait(j, b)
            compute([bufs[b] for _, bufs, _s in streams], j)
        return 0

    assert n_chunks % NBUF == 0
    lax.fori_loop(0, n_chunks // NBUF, outer, 0)


def _agg_chunks(src_hbm, dst_hbm, src_bufs, dst_bufs, sem_s, sem_d,
                nA, baseA, baseB, hsA_v, hsB_v, accA_v, accB_v):
    def base_of(j):
        return jnp.where(j < nA, baseA + j * CH, baseB + (j - nA) * CH)

    def compute(bufs, j):
        sref, dref = bufs

        @pl.when(j < nA)
        def _():
            @plsc.parallel_loop(0, CH, L, unroll=UNROLL)
            def _(o):
                s16 = sref[pl.ds(o, L)]
                d16 = dref[pl.ds(o, L)]
                hv = plsc.load_gather(hsA_v, [s16])
                plsc.addupdate_scatter(accA_v, [d16], hv)

        @pl.when(j >= nA)
        def _():
            @plsc.parallel_loop(0, CH, L, unroll=UNROLL)
            def _(o):
                s16 = sref[pl.ds(o, L)]
                d16 = dref[pl.ds(o, L)]
                hv = plsc.load_gather(hsB_v, [s16])
                plsc.addupdate_scatter(accB_v, [d16], hv)

    _ring(NTCH, base_of,
          ((src_hbm, src_bufs, sem_s), (dst_hbm, dst_bufs, sem_d)), compute)


def _sc_body(hT_hbm, src_hbm, dst_hbm, b_hbm, qT_hbm, part_hbm,
             dinv_v, hsA_v, hsB_v, accA_v, accB_v,
             src_v0, src_v1, dst_v0, dst_v1, b_v,
             sem_s0, sem_s1, sem_d0, sem_d1, sem_hA, sem_hB):
    src_bufs = (src_v0, src_v1)
    dst_bufs = (dst_v0, dst_v1)
    sem_s = (sem_s0, sem_s1)
    sem_d = (sem_d0, sem_d1)
    cid = lax.axis_index("c")
    t = lax.axis_index("s")

    p = t // 4
    r = t % 4
    i32 = jnp.int32
    fA_l = 3 * p + jnp.where(r <= 1, 0, jnp.where(r == 2, 1, 2))
    fB_l = jnp.where(r == 1, 3 * p + 1, jnp.where(r == 2, 3 * p + 2, fA_l))
    baseA = jnp.where(r == 0, 0,
                      jnp.where(r == 1, 120000,
                                jnp.where(r == 2, 80000, 40000))).astype(i32)
    nA = jnp.where((r == 0) | (r == 3), 30,
                   jnp.where(r == 1, 10, 20)).astype(i32)
    baseB = i32(0)
    owns = r != 1
    fA = FSC * cid + fA_l
    fB = FSC * cid + fB_l
    f_own = fA

    cpA = pltpu.make_async_copy(hT_hbm.at[fA], hsA_v, sem_hA)
    cpB = pltpu.make_async_copy(hT_hbm.at[fB], hsB_v, sem_hB)
    cpA.start()
    cpB.start()
    pltpu.sync_copy(b_hbm, b_v)

    @plsc.parallel_loop(0, N_NODES_C, L, unroll=4)
    def _(o):
        dinv_v[pl.ds(o, L)] = jnp.full((L,), 1.0, jnp.float32)

    ones = jnp.full((L,), 1.0, jnp.float32)

    def degcompute(bufs, j):
        dref = bufs[0]

        @plsc.parallel_loop(0, CH, L, unroll=UNROLL)
        def _(o):
            d16 = dref[pl.ds(o, L)]
            plsc.addupdate_scatter(dinv_v, [d16], ones)

    _ring(NCHUNK_ALL, lambda j: j * CH,
          ((dst_hbm, dst_bufs, sem_d),), degcompute)

    @plsc.parallel_loop(0, N_NODES_C, L, unroll=4)
    def _(o):
        s = pl.ds(o, L)
        dinv_v[s] = _rsqrt16(dinv_v[s])

    cpA.wait()
    cpB.wait()

    ownf = owns.astype(jnp.float32)

    @plsc.parallel_loop(0, N_NODES_C, L, unroll=4)
    def _(o):
        s = pl.ds(o, L)
        hA = hsA_v[s] * dinv_v[s]
        hsA_v[s] = hA
        hsB_v[s] = hsB_v[s] * dinv_v[s]
        accA_v[s] = hA * ownf
        accB_v[s] = jnp.zeros((L,), jnp.float32)

    _agg_chunks(src_hbm, dst_hbm, src_bufs, dst_bufs, sem_s, sem_d,
                nA, baseA, baseB, hsA_v, hsB_v, accA_v, accB_v)

    @pl.when(r == 1)
    def _():
        pltpu.sync_copy(accA_v, part_hbm.at[FSC * cid + 3 * p])
        pltpu.sync_copy(accB_v, part_hbm.at[FSC * cid + 3 * p + 1])

    @pl.when(r == 2)
    def _():
        pltpu.sync_copy(accB_v, part_hbm.at[FSC * cid + 3 * p + 2])

    plsc.subcore_barrier()

    bvec = plsc.load_gather(b_v, [jnp.full((L,), f_own, jnp.int32)])

    @pl.when(owns)
    def _():
        pltpu.sync_copy(part_hbm.at[fA], accB_v)

        @plsc.parallel_loop(0, N_NODES_C, L, unroll=4)
        def _(o):
            s = pl.ds(o, L)
            z = (accA_v[s] + accB_v[s]) * dinv_v[s] + bvec
            accA_v[s] = jnp.maximum(z, 0.0)

        pltpu.sync_copy(accA_v, part_hbm.at[fA])

    plsc.subcore_barrier()

    pltpu.sync_copy(part_hbm.at[fA], hsA_v)
    pltpu.sync_copy(part_hbm.at[fB], hsB_v)
    plsc.subcore_barrier()

    @plsc.parallel_loop(0, N_NODES_C, L, unroll=4)
    def _(o):
        s = pl.ds(o, L)
        hA = hsA_v[s] * dinv_v[s]
        hsA_v[s] = hA
        hsB_v[s] = hsB_v[s] * dinv_v[s]
        accA_v[s] = hA * ownf
        accB_v[s] = jnp.zeros((L,), jnp.float32)

    _agg_chunks(src_hbm, dst_hbm, src_bufs, dst_bufs, sem_s, sem_d,
                nA, baseA, baseB, hsA_v, hsB_v, accA_v, accB_v)

    @pl.when(r == 1)
    def _():
        pltpu.sync_copy(accA_v, part_hbm.at[FSC * cid + 3 * p])
        pltpu.sync_copy(accB_v, part_hbm.at[FSC * cid + 3 * p + 1])

    @pl.when(r == 2)
    def _():
        pltpu.sync_copy(accB_v, part_hbm.at[FSC * cid + 3 * p + 2])

    plsc.subcore_barrier()

    @pl.when(owns)
    def _():
        pltpu.sync_copy(part_hbm.at[fA], accB_v)

        @plsc.parallel_loop(0, N_NODES_C, L, unroll=4)
        def _(o):
            s = pl.ds(o, L)
            accA_v[s] = (accA_v[s] + accB_v[s]) * dinv_v[s]

        pltpu.sync_copy(accA_v, qT_hbm.at[f_own])


def _sc_gcn(hT, src, dst, bpad):
    f = pl.kernel(
        _sc_body,
        out_type=(
            jax.ShapeDtypeStruct((2 * FSC, N_NODES_C), jnp.float32),
            jax.ShapeDtypeStruct((2 * FSC, N_NODES_C), jnp.float32),
        ),
        mesh=_sc_mesh(),
        scratch_types=[
            pltpu.VMEM((N_NODES_C,), jnp.float32),
            pltpu.VMEM((N_NODES_C,), jnp.float32),
            pltpu.VMEM((N_NODES_C,), jnp.float32),
            pltpu.VMEM((N_NODES_C,), jnp.float32),
            pltpu.VMEM((N_NODES_C,), jnp.float32),
            pltpu.VMEM((CH,), jnp.int32),
            pltpu.VMEM((CH,), jnp.int32),
            pltpu.VMEM((CH,), jnp.int32),
            pltpu.VMEM((CH,), jnp.int32),
            pltpu.VMEM((32,), jnp.float32),
            pltpu.SemaphoreType.DMA,
            pltpu.SemaphoreType.DMA,
            pltpu.SemaphoreType.DMA,
            pltpu.SemaphoreType.DMA,
            pltpu.SemaphoreType.DMA,
            pltpu.SemaphoreType.DMA,
        ],
        compiler_params=pltpu.CompilerParams(needs_layout_passes=False),
    )
    return f(hT, src, dst, bpad)[0]


def _mm_body(x_ref, w_ref, o_ref):
    o_ref[...] = jnp.dot(x_ref[...], w_ref[...],
                         preferred_element_type=jnp.float32)


def _tc_matmul(x, w):
    return pl.pallas_call(
        _mm_body,
        out_shape=jax.ShapeDtypeStruct((x.shape[0], w.shape[1]), jnp.float32),
    )(x, w)


def _tail_body(q_ref, w2_ref, b2_ref, wfc_ref, bfc_ref, o_ref):
    z2 = jnp.maximum(
        jnp.dot(q_ref[...], w2_ref[...], preferred_element_type=jnp.float32)
        + b2_ref[...],
        0.0,
    )
    o_ref[...] = jnp.maximum(
        jnp.dot(z2, wfc_ref[...], preferred_element_type=jnp.float32)
        + bfc_ref[...],
        0.0,
    )


def _tc_tail(q, W2, b2, Wfc, bfc):
    return pl.pallas_call(
        _tail_body,
        out_shape=jax.ShapeDtypeStruct((q.shape[0], Wfc.shape[1]),
                                       jnp.float32),
    )(q, W2, b2.reshape(1, -1), Wfc, bfc.reshape(1, -1))


def kernel(x, edge_index, W1, b1, W2, b2, Wfc, bfc):
    ei = edge_index.astype(jnp.int32)
    src = ei[0]
    dst = ei[1]
    b1p = jnp.zeros((32,), jnp.float32).at[: b1.shape[0]].set(b1)

    h1 = _tc_matmul(x, W1)
    qT = _sc_gcn(h1.T, src, dst, b1p)
    out = _tc_tail(qT.T, W2, b2, Wfc, bfc)
    return out
